# baseline (speedup 1.0000x reference)
import jax
import jax.numpy as jnp
import numpy as np
from jax import lax
from jax.experimental import pallas as pl
from jax.experimental.pallas import tpu as pltpu
from jax.experimental.pallas import tpu_sc as plsc

_N, _C, _H, _W = 32, 10, 30, 30
_PAD = 4
_HP = _H + 2 * _PAD
_CH_STRIDE = _HP * _HP
_XP_SIZE = _C * _CH_STRIDE
_XP_ALLOC = 14448
_ROW = 810
_NPIX = _C * _H * _W
_NPIX_PAD = 9008
_RPI = _H * _W
_OFFS = (0, 16, 32, 48, 64, 65)


def _build_tables():
    q = np.arange(_NPIX_PAD)
    c = q // _RPI
    r = (q // _W) % _H
    col = q % _W
    scat = c * _CH_STRIDE + (r + _PAD) * _HP + (col + _PAD)
    scat[_NPIX:] = _XP_SIZE + np.arange(_NPIX_PAD - _NPIX)
    p = np.arange(_ROW)
    c2 = p // 81
    k = p % 81
    i = k // 9
    j = k % 9
    rowstatic = c2 * _CH_STRIDE + i * _HP + j
    jj = np.arange(904) % _RPI
    comb = np.broadcast_to((jj // _W) * _HP + (jj % _W),
                           (_N, 904)).reshape(-1).copy()
    return (jnp.asarray(scat, jnp.int32), jnp.asarray(rowstatic, jnp.int32),
            jnp.asarray(comb, jnp.int32))


def _sc_body(x_hbm, scat_hbm, rs_hbm, comb_hbm, out_hbm,
             xp, stage, scat_v, rs_v, comb_v,
             b0, b1, b2, b3, s0, s1, s2, s3):
    n = lax.axis_index("s") * 2 + lax.axis_index("c")
    start = n * _RPI
    bufs = (b0, b1, b2, b3)
    sems = (s0, s1, s2, s3)

    pltpu.sync_copy(scat_hbm, scat_v)
    pltpu.sync_copy(rs_hbm, rs_v)
    pltpu.sync_copy(comb_hbm.at[pl.ds(pl.multiple_of(n * 904, 8), 904)],
                    comb_v.at[pl.ds(0, 904)])
    pltpu.sync_copy(x_hbm.at[pl.ds(pl.multiple_of(n * _NPIX, 8), _NPIX)],
                    stage.at[pl.ds(0, _NPIX)])

    ones = jnp.full((16,), 1.0, jnp.float32)
    zeros = jnp.zeros((16,), jnp.float32)

    def fill_ones(t, carry):
        xp[pl.ds(16 * t, 16)] = ones
        return carry

    lax.fori_loop(0, 91, fill_ones, 0)

    def fill_zeros(t, carry):
        xp[pl.ds(1456 + 16 * t, 16)] = zeros
        return carry

    lax.fori_loop(0, 812, fill_zeros, 0)
    seam = lax.iota(jnp.int32, 16) + 1444
    plsc.store_scatter(xp, [seam], zeros)

    def scatter_in(t, carry):
        vals = stage[pl.ds(16 * t, 16)]
        idx = scat_v[pl.ds(16 * t, 16)]
        plsc.store_scatter(xp, [idx], vals)
        return carry

    lax.fori_loop(0, _NPIX_PAD // 16, scatter_in, 0)

    def fill_pair(p, g):
        combs = comb_v[pl.ds(2 * g, 16)]
        offs = [jnp.full((16,), combs[b], jnp.int32) for b in range(2)]

        @plsc.parallel_loop(0, _C, 1)
        def _(c):
            for o in _OFFS:
                rsv = rs_v[pl.ds(81 * c + o, 16)]
                for b in range(2):
                    bufs[2 * p + b][c, pl.ds(o, 16)] = plsc.load_gather(
                        xp, [rsv + offs[b]])
        for b in range(2):
            pltpu.async_copy(bufs[2 * p + b],
                             out_hbm.at[start + 2 * g + b], sems[2 * p + b])

    def wait_pair(p):
        for b in range(2):
            pltpu.make_async_copy(bufs[2 * p + b], out_hbm.at[0],
                                  sems[2 * p + b]).wait()

    fill_pair(0, 0)
    fill_pair(1, 1)

    def gbody(m, carry):
        wait_pair(0)
        fill_pair(0, 2 * m)
        wait_pair(1)
        fill_pair(1, 2 * m + 1)
        return carry

    lax.fori_loop(1, _RPI // 4, gbody, 0)
    wait_pair(0)
    wait_pair(1)


@jax.jit
def kernel(x):
    scat, rowstatic, comb = _build_tables()
    x1d = x.reshape(_N * _NPIX)

    run = pl.kernel(
        _sc_body,
        out_type=jax.ShapeDtypeStruct((_N * _RPI, _C, 81), jnp.float32),
        mesh=plsc.VectorSubcoreMesh(core_axis_name="c", subcore_axis_name="s"),
        compiler_params=pltpu.CompilerParams(needs_layout_passes=False),
        scratch_types=[
            pltpu.VMEM((_XP_ALLOC,), jnp.float32),
            pltpu.VMEM((_NPIX_PAD,), jnp.float32),
            pltpu.VMEM((_NPIX_PAD,), jnp.int32),
            pltpu.VMEM((_ROW,), jnp.int32),
            pltpu.VMEM((920,), jnp.int32),
            pltpu.VMEM((_C, 81), jnp.float32),
            pltpu.VMEM((_C, 81), jnp.float32),
            pltpu.VMEM((_C, 81), jnp.float32),
            pltpu.VMEM((_C, 81), jnp.float32),
            pltpu.SemaphoreType.DMA,
            pltpu.SemaphoreType.DMA,
            pltpu.SemaphoreType.DMA,
            pltpu.SemaphoreType.DMA,
        ],
    )
    return run(x1d, scat, rowstatic, comb)


# speedup vs baseline: 2.4576x; 2.4576x over previous
import jax
import jax.numpy as jnp
import numpy as np
from jax import lax
from jax.experimental import pallas as pl
from jax.experimental.pallas import tpu as pltpu
from jax.experimental.pallas import tpu_sc as plsc

_N, _C, _H, _W = 32, 10, 30, 30
_PAD = 4
_HP = _H + 2 * _PAD
_CH_STRIDE = _HP * _HP
_XP_SIZE = _C * _CH_STRIDE
_XP_IMG = 14448
_NPIX = _C * _H * _W
_NPIX_PAD = 9008
_RPI = _H * _W
_ROWS = _N * _RPI
_BAND = 896


def _build_tables():
    q = np.arange(_NPIX_PAD)
    c = q // _RPI
    r = (q // _W) % _H
    col = q % _W
    scat = c * _CH_STRIDE + (r + _PAD) * _HP + (col + _PAD)
    scat[_NPIX:] = _XP_SIZE + np.arange(_NPIX_PAD - _NPIX)
    rr = np.arange(_ROWS + 128)
    comb = (rr // _RPI) * _XP_IMG + ((rr % _RPI) // _W) * _HP + (rr % _W)
    return (jnp.asarray(scat, jnp.int32), jnp.asarray(comb, jnp.int32))


def _sc_body(x_hbm, scat_hbm, comb_hbm, out_hbm,
             xp, stage, scat_v, comb_v, chunk_a, chunk_b, sem_a, sem_b):
    wid = lax.axis_index("s") * 2 + lax.axis_index("c")
    band0 = wid * _BAND
    img0 = jnp.maximum(wid - 1, 0)
    img1 = jnp.minimum(img0 + 1, _N - 1)
    ntiles = jnp.where(wid == _N - 1, 8, 7)

    pltpu.sync_copy(scat_hbm, scat_v)
    pltpu.sync_copy(comb_hbm.at[pl.ds(pl.multiple_of(band0, 8), 1024)],
                    comb_v)

    ones = jnp.full((16,), 1.0, jnp.float32)
    zeros = jnp.zeros((16,), jnp.float32)

    def fill_ones(t, carry):
        xp[pl.ds(16 * t, 16)] = ones
        xp[pl.ds(_XP_IMG + 16 * t, 16)] = ones
        return carry

    lax.fori_loop(0, 91, fill_ones, 0)

    def fill_zeros(t, carry):
        xp[pl.ds(1456 + 16 * t, 16)] = zeros
        xp[pl.ds(_XP_IMG + 1456 + 16 * t, 16)] = zeros
        return carry

    lax.fori_loop(0, 812, fill_zeros, 0)
    seam = lax.iota(jnp.int32, 16) + 1444
    plsc.store_scatter(xp, [seam], zeros)
    plsc.store_scatter(xp, [seam + _XP_IMG], zeros)

    def stage_image(img_idx, xp_base):
        pltpu.sync_copy(
            x_hbm.at[pl.ds(pl.multiple_of(img_idx * _NPIX, 8), _NPIX)],
            stage.at[pl.ds(0, _NPIX)])

        def scatter_in(t, carry):
            vals = stage[pl.ds(16 * t, 16)]
            idx = scat_v[pl.ds(16 * t, 16)] + xp_base
            plsc.store_scatter(xp, [idx], vals)
            return carry

        lax.fori_loop(0, _NPIX_PAD // 16, scatter_in, 0)

    stage_image(img0, 0)
    stage_image(img1, _XP_IMG)

    base_sub = jnp.full((16,), img0 * _XP_IMG, jnp.int32)

    def rebase(t, carry):
        comb_v[pl.ds(16 * t, 16)] = comb_v[pl.ds(16 * t, 16)] - base_sub
        return carry

    lax.fori_loop(0, 64, rebase, 0)

    def fill_chunk(chunk, sem, c, rt):
        cbase = jnp.full((16,), c * _CH_STRIDE, jnp.int32)

        @plsc.parallel_loop(0, 8, 1)
        def _(g):
            combg = comb_v[pl.ds(128 * rt + 16 * g, 16)] + cbase
            for kk in range(81):
                chunk[kk, pl.ds(16 * g, 16)] = plsc.load_gather(
                    xp, [combg + ((kk // 9) * _HP + kk % 9)])

        off = pl.multiple_of(band0 + 128 * rt, 128)
        pltpu.async_copy(chunk, out_hbm.at[c, :, pl.ds(off, 128)], sem)

    def wait(chunk, sem):
        pltpu.make_async_copy(chunk, out_hbm.at[0, :, pl.ds(0, 128)],
                              sem).wait()

    fill_chunk(chunk_a, sem_a, 0, 0)
    fill_chunk(chunk_b, sem_b, 1, 0)

    def pbody(p, carry):
        rt, cp = carry
        wait(chunk_a, sem_a)
        fill_chunk(chunk_a, sem_a, 2 * cp, rt)
        wait(chunk_b, sem_b)
        fill_chunk(chunk_b, sem_b, 2 * cp + 1, rt)
        wrap = cp == 4
        return (rt + wrap, jnp.where(wrap, 0, cp + 1))

    lax.fori_loop(1, 5 * ntiles, pbody,
                  (jnp.int32(0), jnp.int32(1)))
    wait(chunk_a, sem_a)
    wait(chunk_b, sem_b)


@jax.jit
def kernel(x):
    scat, comb = _build_tables()
    x1d = x.reshape(_N * _NPIX)

    run = pl.kernel(
        _sc_body,
        out_type=jax.ShapeDtypeStruct((_C, 81, _ROWS), jnp.float32),
        mesh=plsc.VectorSubcoreMesh(core_axis_name="c", subcore_axis_name="s"),
        compiler_params=pltpu.CompilerParams(needs_layout_passes=False),
        scratch_types=[
            pltpu.VMEM((2 * _XP_IMG,), jnp.float32),
            pltpu.VMEM((_NPIX_PAD,), jnp.float32),
            pltpu.VMEM((_NPIX_PAD,), jnp.int32),
            pltpu.VMEM((1024,), jnp.int32),
            pltpu.VMEM((81, 128), jnp.float32),
            pltpu.VMEM((81, 128), jnp.float32),
            pltpu.SemaphoreType.DMA,
            pltpu.SemaphoreType.DMA,
        ],
    )
    out_t = run(x1d, scat, comb)
    return jnp.transpose(out_t, (2, 0, 1))


# speedup vs baseline: 3.1165x; 1.2681x over previous
import jax
import jax.numpy as jnp
import numpy as np
from jax import lax
from jax.experimental import pallas as pl
from jax.experimental.pallas import tpu as pltpu
from jax.experimental.pallas import tpu_sc as plsc

_N, _C, _H, _W = 32, 10, 30, 30
_PAD = 4
_HP = _H + 2 * _PAD
_CH_STRIDE = _HP * _HP
_XP_SIZE = _C * _CH_STRIDE
_XP_IMG = 14448
_NPIX = _C * _H * _W
_NPIX_PAD = 9008
_RPI = _H * _W
_ROWS = _N * _RPI
_BAND = 896


def _build_tables():
    q = np.arange(_NPIX_PAD)
    c = q // _RPI
    r = (q // _W) % _H
    col = q % _W
    scat = c * _CH_STRIDE + (r + _PAD) * _HP + (col + _PAD)
    scat[_NPIX:] = _XP_SIZE + np.arange(_NPIX_PAD - _NPIX)
    rr = np.arange(_ROWS + 128)
    comb = (rr // _RPI) * _XP_IMG + ((rr % _RPI) // _W) * _HP + (rr % _W)
    return (jnp.asarray(scat, jnp.int32), jnp.asarray(comb, jnp.int32))


def _sc_body(x_hbm, scat_hbm, comb_hbm, out_hbm,
             xp, stage, scat_v, comb_v, chunk_a, chunk_b, sem_a, sem_b):
    wid = lax.axis_index("s") * 2 + lax.axis_index("c")
    band0 = wid * _BAND
    img0 = jnp.maximum(wid - 1, 0)
    img1 = jnp.minimum(img0 + 1, _N - 1)

    pltpu.sync_copy(scat_hbm, scat_v)
    pltpu.sync_copy(comb_hbm.at[pl.ds(pl.multiple_of(band0, 8), 1024)],
                    comb_v.at[pl.ds(0, 1024)])
    pltpu.sync_copy(comb_hbm.at[pl.ds(_ROWS - 128, 128)],
                    comb_v.at[pl.ds(1024, 128)])

    ones = jnp.full((16,), 1.0, jnp.float32)
    zeros = jnp.zeros((16,), jnp.float32)

    def fill_ones(t, carry):
        xp[pl.ds(16 * t, 16)] = ones
        xp[pl.ds(_XP_IMG + 16 * t, 16)] = ones
        xp[pl.ds(2 * _XP_IMG + 16 * t, 16)] = ones
        return carry

    lax.fori_loop(0, 91, fill_ones, 0)

    def fill_zeros(t, carry):
        xp[pl.ds(1456 + 16 * t, 16)] = zeros
        xp[pl.ds(_XP_IMG + 1456 + 16 * t, 16)] = zeros
        xp[pl.ds(2 * _XP_IMG + 1456 + 16 * t, 16)] = zeros
        return carry

    lax.fori_loop(0, 812, fill_zeros, 0)
    seam = lax.iota(jnp.int32, 16) + 1444
    plsc.store_scatter(xp, [seam], zeros)
    plsc.store_scatter(xp, [seam + _XP_IMG], zeros)
    plsc.store_scatter(xp, [seam + 2 * _XP_IMG], zeros)

    def stage_image(img_idx, xp_base):
        pltpu.sync_copy(
            x_hbm.at[pl.ds(pl.multiple_of(img_idx * _NPIX, 8), _NPIX)],
            stage.at[pl.ds(0, _NPIX)])

        def scatter_in(t, carry):
            vals = stage[pl.ds(16 * t, 16)]
            idx = scat_v[pl.ds(16 * t, 16)] + xp_base
            plsc.store_scatter(xp, [idx], vals)
            return carry

        lax.fori_loop(0, _NPIX_PAD // 16, scatter_in, 0)

    stage_image(img0, 0)
    stage_image(img1, _XP_IMG)
    stage_image(jnp.int32(_N - 1), 2 * _XP_IMG)

    base_sub = jnp.full((16,), img0 * _XP_IMG, jnp.int32)

    def rebase(t, carry):
        comb_v[pl.ds(16 * t, 16)] = comb_v[pl.ds(16 * t, 16)] - base_sub
        return carry

    lax.fori_loop(0, 64, rebase, 0)
    base_sub2 = jnp.full((16,), (_N - 3) * _XP_IMG, jnp.int32)

    def rebase2(t, carry):
        comb_v[pl.ds(1024 + 16 * t, 16)] = (
            comb_v[pl.ds(1024 + 16 * t, 16)] - base_sub2)
        return carry

    lax.fori_loop(0, 8, rebase2, 0)

    def fill_chunk(chunk, sem, c, kt, nk, rw, roff):
        cbase = jnp.full((16,), c * _CH_STRIDE, jnp.int32)

        @plsc.parallel_loop(0, rw // 16, 1)
        def _(g):
            combg = comb_v[pl.ds(roff + 16 * g, 16)] + cbase
            for kk in range(nk):
                k = 8 * kt + kk
                chunk[kk, pl.ds(16 * g, 16)] = plsc.load_gather(
                    xp, [combg + ((k // 9) * _HP + k % 9)])

        dst = out_hbm.at[c, pl.ds(8 * kt, nk),
                         pl.ds(pl.multiple_of(band0 + roff, 128), rw)]
        pltpu.async_copy(chunk.at[pl.ds(0, nk), pl.ds(0, rw)], dst, sem)

    def wait(sem, nk, rw):
        pltpu.make_async_copy(
            chunk_a.at[pl.ds(0, nk), pl.ds(0, rw)],
            out_hbm.at[0, pl.ds(0, nk), pl.ds(0, rw)], sem).wait()

    def run_tiles(rw, roff):
        fill_chunk(chunk_a, sem_a, 0, 0, 8, rw, roff)
        fill_chunk(chunk_b, sem_b, 1, 0, 8, rw, roff)
        for kt in range(11):
            nk = 1 if kt == 10 else 8
            cp_lo = 1 if kt == 0 else 0
            if kt == 10:
                wait(sem_a, 8, rw)
                wait(sem_b, 8, rw)

            def cbody(cp, carry, kt=kt, nk=nk):
                ca, cb = 2 * cp, 2 * cp + 1
                if kt == 10:
                    pass
                else:
                    wait(sem_a, nk, rw)
                fill_chunk(chunk_a, sem_a, ca, kt, nk, rw, roff)
                if kt == 10:
                    wait(sem_a, nk, rw)
                else:
                    wait(sem_b, nk, rw)
                fill_chunk(chunk_b, sem_b, cb, kt, nk, rw, roff)
                if kt == 10:
                    wait(sem_b, nk, rw)
                return carry

            lax.fori_loop(cp_lo, 5, cbody, 0)

    run_tiles(896, 0)

    @pl.when(wid < _C)
    def _():
        cbase = jnp.full((16,), wid * _CH_STRIDE, jnp.int32)
        for kt in range(11):
            nk = 1 if kt == 10 else 8

            @plsc.parallel_loop(0, 8, 1)
            def _(g, kt=kt, nk=nk):
                combg = comb_v[pl.ds(1024 + 16 * g, 16)] + cbase
                for kk in range(nk):
                    k = 8 * kt + kk
                    chunk_a[kk, pl.ds(16 * g, 16)] = plsc.load_gather(
                        xp, [combg + ((k // 9) * _HP + k % 9)])

            dst = out_hbm.at[wid, pl.ds(8 * kt, nk),
                             pl.ds(_ROWS - 128, 128)]
            pltpu.async_copy(chunk_a.at[pl.ds(0, nk), pl.ds(0, 128)],
                             dst, sem_a)
            pltpu.make_async_copy(
                chunk_a.at[pl.ds(0, nk), pl.ds(0, 128)],
                out_hbm.at[0, pl.ds(0, nk), pl.ds(0, 128)], sem_a).wait()


@jax.jit
def kernel(x):
    scat, comb = _build_tables()
    x1d = x.reshape(_N * _NPIX)

    run = pl.kernel(
        _sc_body,
        out_type=jax.ShapeDtypeStruct((_C, 81, _ROWS), jnp.float32),
        mesh=plsc.VectorSubcoreMesh(core_axis_name="c", subcore_axis_name="s"),
        compiler_params=pltpu.CompilerParams(needs_layout_passes=False),
        scratch_types=[
            pltpu.VMEM((3 * _XP_IMG,), jnp.float32),
            pltpu.VMEM((_NPIX_PAD,), jnp.float32),
            pltpu.VMEM((_NPIX_PAD,), jnp.int32),
            pltpu.VMEM((1152,), jnp.int32),
            pltpu.VMEM((8, 896), jnp.float32),
            pltpu.VMEM((8, 896), jnp.float32),
            pltpu.SemaphoreType.DMA,
            pltpu.SemaphoreType.DMA,
        ],
    )
    out_t = run(x1d, scat, comb)
    return jnp.transpose(out_t, (2, 0, 1))
